# Initial kernel scaffold; baseline (speedup 1.0000x reference)
import jax
import jax.numpy as jnp
from jax.experimental import pallas as pl


def kernel(x, edge_index, edge_attr, batch, W_rel1, b_rel1, W_root1, bn1_g, bn1_b, p1, W_rel2, b_rel2, W_root2, bn2_g, bn2_b, p2, W_lin, b_lin):
    raise NotImplementedError("write your pallas kernel here")


# speedup vs baseline: 10.5780x; 10.5780x over previous
import functools
import jax
import jax.numpy as jnp
from jax import lax
from jax.experimental import pallas as pl
from jax.experimental.pallas import tpu as pltpu
from jax.experimental.pallas import tpu_sc as plsc

_N = 10000
_E = 320000
_D_IN = 128
_D_H = 256
_N_PAD = 10240
_NC, _NS = 2, 16
_NW = _NC * _NS
_CHUNK = 128
_CHUNKS = 79
_E_PAD = _NW * _CHUNKS * _CHUNK
_K1 = 5000
_K2 = 2500
_ROWS_PER_TILE = _N_PAD // _NS

_SQRT1_2 = 0.7071067811865476


def _gelu(x):
    return x * 0.5 * (1.0 + lax.erf(x * _SQRT1_2))


def _topk_mask(score, valid, k):
    i = lax.bitcast_convert_type(score, jnp.int32)
    srt = jnp.bitwise_xor(
        i, jnp.bitwise_and(lax.shift_right_arithmetic(i, 31), jnp.int32(0x7FFFFFFF)))
    u = lax.bitcast_convert_type(
        jnp.bitwise_xor(srt, jnp.int32(-0x80000000)), jnp.uint32)
    u = jnp.where(valid, u, jnp.uint32(0))
    t = jnp.uint32(0)
    for b in range(31, -1, -1):
        cand = jnp.bitwise_or(t, jnp.uint32(1 << b))
        cnt = jnp.sum((u >= cand).astype(jnp.int32))
        t = jnp.where(cnt >= k, cand, t)
    tau = t
    cgt = jnp.sum((u > tau).astype(jnp.int32))
    d = k - cgt
    idx = lax.broadcasted_iota(jnp.int32, score.shape, 0)
    tie = u == tau
    g = jnp.int32(0)
    for b in range(13, -1, -1):
        cand = jnp.bitwise_or(g, jnp.int32(1 << b))
        cnt = jnp.sum((tie & (idx < cand)).astype(jnp.int32))
        g = jnp.where(cnt <= d - 1, cand, g)
    return (u > tau) | (tie & (idx <= g))


def _sc_edge_scatter(table, src3, dst3, zeros):
    mesh = plsc.VectorSubcoreMesh(core_axis_name="c", subcore_axis_name="s")

    @functools.partial(
        pl.kernel,
        mesh=mesh,
        out_type=jax.ShapeDtypeStruct((_NC, _N_PAD, 128), jnp.float32),
        scratch_types=[
            pltpu.VMEM((_CHUNKS, _CHUNK), jnp.int32),
            pltpu.VMEM((_CHUNKS, _CHUNK), jnp.int32),
            pltpu.VMEM((_CHUNK, 128), jnp.float32),
            pltpu.VMEM_SHARED((_N_PAD, 128), jnp.float32),
            pltpu.SemaphoreType.DMA,
        ],
    )
    def k(table_hbm, src_hbm, dst_hbm, zero_hbm, out_hbm,
          src_v, dst_v, rows_v, agg_sh, sem):
        c = lax.axis_index("c")
        s = lax.axis_index("s")
        wid = s * _NC + c
        r0 = s * _ROWS_PER_TILE
        pltpu.sync_copy(zero_hbm.at[pl.ds(r0, _ROWS_PER_TILE)],
                        agg_sh.at[pl.ds(r0, _ROWS_PER_TILE)])
        pltpu.sync_copy(src_hbm.at[wid], src_v)
        pltpu.sync_copy(dst_hbm.at[wid], dst_v)
        plsc.subcore_barrier()

        def step(j, carry):
            pltpu.async_copy(table_hbm.at[src_v.at[j]], rows_v, sem).wait()
            pltpu.sync_copy(rows_v, agg_sh.at[dst_v.at[j]], add=True)
            return carry

        lax.fori_loop(0, _CHUNKS, step, 0)
        plsc.subcore_barrier()
        pltpu.sync_copy(agg_sh.at[pl.ds(r0, _ROWS_PER_TILE)],
                        out_hbm.at[c].at[pl.ds(r0, _ROWS_PER_TILE)])

    return k(table, src3, dst3, zeros)


def _tc1a_body(p_ref, x_ref, wr1_ref, br1_ref, wq1_ref, g1_ref, b1_ref,
               p1_ref, h_ref, score_ref):
    agg = p_ref[0] + p_ref[1]
    x = x_ref[...]
    h = (jnp.dot(agg, wr1_ref[...], preferred_element_type=jnp.float32)
         + jnp.dot(x, wq1_ref[...], preferred_element_type=jnp.float32)
         + br1_ref[...])
    h = _gelu(h)
    rowm = (lax.broadcasted_iota(jnp.int32, (_N_PAD, 1), 0) < _N)
    rowmf = rowm.astype(jnp.float32)
    inv_n = jnp.float32(1.0 / _N)
    mean = jnp.sum(h * rowmf, axis=0, keepdims=True) * inv_n
    var = jnp.sum(((h - mean) ** 2) * rowmf, axis=0, keepdims=True) * inv_n
    h = (h - mean) * lax.rsqrt(var + 1e-5) * g1_ref[...] + b1_ref[...]
    p1 = p1_ref[...]
    pn = lax.rsqrt(jnp.sum(p1 * p1))
    score_ref[...] = jnp.tanh(
        lax.dot_general(h, p1, (((1,), (1,)), ((), ())),
                        preferred_element_type=jnp.float32) * pn)
    h_ref[...] = h


def _tc1b_body(h_ref, score_ref, x1_ref, keep_ref):
    h = h_ref[...]
    score = score_ref[...]
    rowm = (lax.broadcasted_iota(jnp.int32, (_N_PAD, 1), 0) < _N)
    mask = _topk_mask(score, rowm, _K1)
    maskf = mask.astype(jnp.float32)
    hs = h * score
    neg = jnp.float32(-3.0e38)
    x1max = jnp.max(jnp.where(mask, hs, neg), axis=0, keepdims=True)
    inv_k = jnp.float32(1.0 / _K1)
    x1mean = jnp.sum(hs * maskf, axis=0, keepdims=True) * inv_k
    x1_ref[...] = jnp.concatenate([x1max, x1mean], axis=1)
    keep_ref[...] = maskf


def _tc1c_body(h_ref, score_ref, keep_ref, x1_ref, g2_ref, b2_ref, t0_ref,
               t1_ref):
    maskf = keep_ref[...]
    hp = h_ref[...] * score_ref[...] * maskf
    inv_k = jnp.float32(1.0 / _K1)
    m2 = x1_ref[0:1, _D_H:2 * _D_H]
    v2 = jnp.sum(((hp - m2) ** 2) * maskf, axis=0, keepdims=True) * inv_k
    hb = (hp - m2) * lax.rsqrt(v2 + 1e-5) * g2_ref[...] + b2_ref[...]
    hb = _gelu(hb) * maskf
    t0_ref[...] = hb[:, :128]
    t1_ref[...] = hb[:, 128:]


def _tc2a_body(q0a_ref, q0b_ref, q1a_ref, q1b_ref, t0_ref, t1_ref, wr2_ref,
               br2_ref, wq2_ref, h2_ref):
    h2 = (jnp.dot(q0a_ref[...] + q0b_ref[...], wr2_ref[0:128, :],
                  preferred_element_type=jnp.float32)
          + jnp.dot(q1a_ref[...] + q1b_ref[...], wr2_ref[128:256, :],
                    preferred_element_type=jnp.float32)
          + jnp.dot(t0_ref[...], wq2_ref[0:128, :],
                    preferred_element_type=jnp.float32)
          + jnp.dot(t1_ref[...], wq2_ref[128:256, :],
                    preferred_element_type=jnp.float32)
          + br2_ref[...])
    h2_ref[...] = h2


def _tc2b_body(h2_ref, keep_ref, x1_ref, p2_ref, wlin_ref, blin_ref, out_ref):
    h2 = h2_ref[...]
    keep = keep_ref[...] > 0
    p2 = p2_ref[...]
    pn = lax.rsqrt(jnp.sum(p2 * p2))
    score = jnp.tanh(
        lax.dot_general(h2, p2, (((1,), (1,)), ((), ())),
                        preferred_element_type=jnp.float32) * pn)
    mask = _topk_mask(score, keep, _K2)
    maskf = mask.astype(jnp.float32)
    g = h2 * score
    neg = jnp.float32(-3.0e38)
    x2max = jnp.max(jnp.where(mask, g, neg), axis=0, keepdims=True)
    x2mean = jnp.sum(g * maskf, axis=0, keepdims=True) * jnp.float32(1.0 / _K2)
    x12 = x1_ref[...] + jnp.concatenate([x2max, x2mean], axis=1)
    out_ref[...] = (jnp.dot(x12, wlin_ref[...],
                            preferred_element_type=jnp.float32)
                    + blin_ref[...])


def _run_tc1(partials, x_pad, W_rel1, b_rel1, W_root1, bn1_g, bn1_b, p1,
             bn2_g, bn2_b):
    h, score = pl.pallas_call(
        _tc1a_body,
        out_shape=[
            jax.ShapeDtypeStruct((_N_PAD, _D_H), jnp.float32),
            jax.ShapeDtypeStruct((_N_PAD, 1), jnp.float32),
        ],
    )(partials, x_pad, W_rel1, b_rel1.reshape(1, -1), W_root1,
      bn1_g.reshape(1, -1), bn1_b.reshape(1, -1), p1.reshape(1, -1))
    x1, keep = pl.pallas_call(
        _tc1b_body,
        out_shape=[
            jax.ShapeDtypeStruct((1, 2 * _D_H), jnp.float32),
            jax.ShapeDtypeStruct((_N_PAD, 1), jnp.float32),
        ],
    )(h, score)
    t0, t1 = pl.pallas_call(
        _tc1c_body,
        out_shape=[
            jax.ShapeDtypeStruct((_N_PAD, 128), jnp.float32),
            jax.ShapeDtypeStruct((_N_PAD, 128), jnp.float32),
        ],
    )(h, score, keep, x1, bn2_g.reshape(1, -1), bn2_b.reshape(1, -1))
    return t0, t1, x1, keep


def _run_tc2(q0, q1, t0, t1, keep, x1, W_rel2, b_rel2, W_root2, p2,
             W_lin, b_lin):
    h2 = pl.pallas_call(
        _tc2a_body,
        out_shape=jax.ShapeDtypeStruct((_N_PAD, _D_H), jnp.float32),
    )(q0[0], q0[1], q1[0], q1[1], t0, t1,
      W_rel2, b_rel2.reshape(1, -1), W_root2)
    return pl.pallas_call(
        _tc2b_body,
        out_shape=jax.ShapeDtypeStruct((1, _D_H), jnp.float32),
    )(h2, keep, x1, p2.reshape(1, -1), W_lin, b_lin.reshape(1, -1))


def kernel(x, edge_index, edge_attr, batch, W_rel1, b_rel1, W_root1, bn1_g,
           bn1_b, p1, W_rel2, b_rel2, W_root2, bn2_g, bn2_b, p2, W_lin, b_lin):
    f32 = jnp.float32
    x_pad = jnp.zeros((_N_PAD, _D_IN), f32).at[:_N].set(x)
    src = edge_index[0].astype(jnp.int32)
    dst = edge_index[1].astype(jnp.int32)
    pad_n = _E_PAD - _E
    pad_idx = _N + (jnp.arange(pad_n, dtype=jnp.int32) % (_N_PAD - _N))
    src3 = jnp.concatenate([src, pad_idx]).reshape(_NW, _CHUNKS, _CHUNK)
    dst3 = jnp.concatenate([dst, pad_idx]).reshape(_NW, _CHUNKS, _CHUNK)
    zeros = jnp.zeros((_N_PAD, 128), f32)

    partials = _sc_edge_scatter(x_pad, src3, dst3, zeros)
    t0, t1, x1, keep = _run_tc1(partials, x_pad, W_rel1, b_rel1, W_root1,
                                bn1_g, bn1_b, p1, bn2_g, bn2_b)
    q0 = _sc_edge_scatter(t0, src3, dst3, zeros)
    q1 = _sc_edge_scatter(t1, src3, dst3, zeros)
    return _run_tc2(q0, q1, t0, t1, keep, x1, W_rel2, b_rel2, W_root2, p2,
                    W_lin, b_lin)
